# Initial kernel scaffold; baseline (speedup 1.0000x reference)
import jax
import jax.numpy as jnp
from jax.experimental import pallas as pl


def kernel(x, edge_index, edge_attr, W1, b1, root1, bias1, W2, b2, root2, bias2):
    raise NotImplementedError("write your pallas kernel here")


# speedup vs baseline: 1.3765x; 1.3765x over previous
import functools

import jax
import jax.numpy as jnp
from jax import lax
from jax.experimental import pallas as pl
from jax.experimental.pallas import tpu as pltpu
from jax.experimental.pallas import tpu_sc as plsc

N = 10000
E = 160000
IN1, OUT1 = 8, 64
IN2, OUT2 = 64, 64

NC, NS = 2, 16
NW = NC * NS
CH = 128
CPW = 40
E_PAD = NW * CH * CPW

BE = 640
BN = 1000


def _mesh():
    return plsc.VectorSubcoreMesh(
        core_axis_name="c", subcore_axis_name="s", num_cores=NC, num_subcores=NS
    )


def _sc_gather(table, idx, d):
    @functools.partial(
        pl.kernel,
        out_type=jax.ShapeDtypeStruct((E_PAD, d), jnp.float32),
        mesh=_mesh(),
        scratch_types=[
            pltpu.VMEM((CH,), jnp.int32),
            pltpu.VMEM((CH, d), jnp.float32),
            pltpu.SemaphoreType.DMA,
        ],
        compiler_params=pltpu.CompilerParams(use_tc_tiling_on_sc=False),
        interpret=False,
    )
    def gk(tab_hbm, idx_hbm, out_hbm, idx_v, rows_v, sem):
        wid = lax.axis_index("s") * NC + lax.axis_index("c")
        for j in range(CPW):
            base = pl.multiple_of((wid * CPW + j) * CH, CH)
            pltpu.sync_copy(idx_hbm.at[pl.ds(base, CH)], idx_v)
            pltpu.async_copy(tab_hbm.at[idx_v], rows_v, sem).wait()
            pltpu.sync_copy(rows_v, out_hbm.at[pl.ds(base, CH)])

    return gk(table, idx)


def _sc_scatter_add(msg, dst, zeros_n):
    @functools.partial(
        pl.kernel,
        out_type=(
            jax.ShapeDtypeStruct((N, 64), jnp.float32),
            jax.ShapeDtypeStruct((N, 64), jnp.float32),
        ),
        mesh=_mesh(),
        scratch_types=[
            pltpu.VMEM((CH,), jnp.int32),
            pltpu.VMEM((CH, 64), jnp.float32),
            pltpu.VMEM_SHARED((N, 64), jnp.float32),
            pltpu.SemaphoreType.DMA,
        ],
        compiler_params=pltpu.CompilerParams(use_tc_tiling_on_sc=False),
        interpret=False,
    )
    def sk(msg_hbm, dst_hbm, z_hbm, out0, out1, idx_v, msg_v, acc, sem):
        c = lax.axis_index("c")
        s = lax.axis_index("s")

        @pl.when(s < 15)
        def _():
            r0 = pl.multiple_of(s * 624, 8)
            pltpu.sync_copy(z_hbm.at[pl.ds(r0, 624)], acc.at[pl.ds(r0, 624)])

        @pl.when(s == 15)
        def _():
            pltpu.sync_copy(z_hbm.at[pl.ds(9360, 640)], acc.at[pl.ds(9360, 640)])

        plsc.subcore_barrier()

        for j in range(CPW):
            base = pl.multiple_of(((s * NC + c) * CPW + j) * CH, CH)
            pltpu.sync_copy(dst_hbm.at[pl.ds(base, CH)], idx_v)
            pltpu.sync_copy(msg_hbm.at[pl.ds(base, CH)], msg_v)
            pltpu.sync_copy(msg_v, acc.at[idx_v], add=True)

        plsc.subcore_barrier()

        def dump(out_hbm):
            @pl.when(s < 15)
            def _():
                r0 = pl.multiple_of(s * 624, 8)
                pltpu.sync_copy(acc.at[pl.ds(r0, 624)], out_hbm.at[pl.ds(r0, 624)])

            @pl.when(s == 15)
            def _():
                pltpu.sync_copy(acc.at[pl.ds(9360, 640)], out_hbm.at[pl.ds(9360, 640)])

        @pl.when(c == 0)
        def _():
            dump(out0)

        @pl.when(c == 1)
        def _():
            dump(out1)

    return sk(msg, dst, zeros_n)


def _tc_edge_msgs(ea_pad, xs_pad, w2d, b_row, p_mat, q_mat, in_c, out_c):
    io = in_c * out_c
    grid = E_PAD // BE
    real_blocks = E // BE

    def body(ea_ref, xs_ref, w_ref, b_ref, p_ref, q_ref, o_ref):
        blk = pl.program_id(0)

        @pl.when(blk < real_blocks)
        def _():
            a = jnp.dot(ea_ref[...], w_ref[...], preferred_element_type=jnp.float32)
            a = jnp.maximum(a + b_ref[...], 0.0)
            xe = jnp.dot(xs_ref[...], p_ref[...], preferred_element_type=jnp.float32)
            o_ref[...] = jnp.dot(a * xe, q_ref[...], preferred_element_type=jnp.float32)

        @pl.when(blk >= real_blocks)
        def _():
            o_ref[...] = jnp.zeros((BE, out_c), jnp.float32)

    return pl.pallas_call(
        body,
        grid=(grid,),
        in_specs=[
            pl.BlockSpec((BE, 2), lambda i: (i, 0)),
            pl.BlockSpec((BE, in_c), lambda i: (i, 0)),
            pl.BlockSpec((2, io), lambda i: (0, 0)),
            pl.BlockSpec((1, io), lambda i: (0, 0)),
            pl.BlockSpec((in_c, io), lambda i: (0, 0)),
            pl.BlockSpec((io, out_c), lambda i: (0, 0)),
        ],
        out_specs=pl.BlockSpec((BE, out_c), lambda i: (i, 0)),
        out_shape=jax.ShapeDtypeStruct((E_PAD, out_c), jnp.float32),
        interpret=False,
    )(ea_pad, xs_pad, w2d, b_row, p_mat, q_mat)


def _tc_combine(p0, p1, x_in, root, bias_row, in_c):
    def body(a_ref, b_ref, x_ref, r_ref, bias_ref, o_ref):
        o_ref[...] = (
            a_ref[...]
            + b_ref[...]
            + bias_ref[...]
            + jnp.dot(x_ref[...], r_ref[...], preferred_element_type=jnp.float32)
        )

    return pl.pallas_call(
        body,
        grid=(N // BN,),
        in_specs=[
            pl.BlockSpec((BN, 64), lambda i: (i, 0)),
            pl.BlockSpec((BN, 64), lambda i: (i, 0)),
            pl.BlockSpec((BN, in_c), lambda i: (i, 0)),
            pl.BlockSpec((in_c, 64), lambda i: (0, 0)),
            pl.BlockSpec((1, 64), lambda i: (0, 0)),
        ],
        out_specs=pl.BlockSpec((BN, 64), lambda i: (i, 0)),
        out_shape=jax.ShapeDtypeStruct((N, 64), jnp.float32),
        interpret=False,
    )(p0, p1, x_in, root, bias_row)


def _sel_mats(in_c, out_c):
    io = in_c * out_c
    j = jnp.arange(io)
    p_mat = (j[None, :] // out_c == jnp.arange(in_c)[:, None]).astype(jnp.float32)
    q_mat = (j[:, None] % out_c == jnp.arange(out_c)[None, :]).astype(jnp.float32)
    return p_mat, q_mat


def _layer(x_in, src_p, dst_p, ea_p, w, b, root, bias, zeros_n, in_c, out_c):
    xs = _sc_gather(x_in, src_p, in_c)
    p_mat, q_mat = _sel_mats(in_c, out_c)
    msg = _tc_edge_msgs(ea_p, xs, w, b.reshape(1, -1), p_mat, q_mat, in_c, out_c)
    part0, part1 = _sc_scatter_add(msg, dst_p, zeros_n)
    return _tc_combine(part0, part1, x_in, root, bias.reshape(1, -1), in_c)


def kernel(x, edge_index, edge_attr, W1, b1, root1, bias1, W2, b2, root2, bias2):
    x = x.astype(jnp.float32)
    ea = edge_attr.astype(jnp.float32)
    src = edge_index[0].astype(jnp.int32)
    dst = edge_index[1].astype(jnp.int32)

    pad = E_PAD - E
    src_p = jnp.concatenate([src, jnp.zeros((pad,), jnp.int32)])
    dst_p = jnp.concatenate([dst, jnp.zeros((pad,), jnp.int32)])
    ea_p = jnp.concatenate([ea, jnp.zeros((pad, 2), jnp.float32)])
    zeros_n = jnp.zeros((N, 64), jnp.float32)

    h = _layer(x, src_p, dst_p, ea_p, W1, b1, root1, bias1, zeros_n, IN1, OUT1)
    out = _layer(h, src_p, dst_p, ea_p, W2, b2, root2, bias2, zeros_n, IN2, OUT2)
    return out


# speedup vs baseline: 1.8604x; 1.3516x over previous
import functools

import jax
import jax.numpy as jnp
from jax import lax
from jax.experimental import pallas as pl
from jax.experimental.pallas import tpu as pltpu
from jax.experimental.pallas import tpu_sc as plsc

N = 10000
E = 160000
IN1, OUT1 = 8, 64
IN2, OUT2 = 64, 64

NC, NS = 2, 16
NW = NC * NS
CH = 128
CPW = 40
E_PAD = NW * CH * CPW

BE = 640
BN = 1000


def _mesh():
    return plsc.VectorSubcoreMesh(
        core_axis_name="c", subcore_axis_name="s", num_cores=NC, num_subcores=NS
    )


def _sc_gather(table, idx, d):
    @functools.partial(
        pl.kernel,
        out_type=jax.ShapeDtypeStruct((E_PAD, d), jnp.float32),
        mesh=_mesh(),
        scratch_types=[
            pltpu.VMEM((CH,), jnp.int32),
            pltpu.VMEM((CH, d), jnp.float32),
            pltpu.SemaphoreType.DMA,
        ],
        compiler_params=pltpu.CompilerParams(use_tc_tiling_on_sc=False),
        interpret=False,
    )
    def gk(tab_hbm, idx_hbm, out_hbm, idx_v, rows_v, sem):
        wid = lax.axis_index("s") * NC + lax.axis_index("c")
        for j in range(CPW):
            base = pl.multiple_of((wid * CPW + j) * CH, CH)
            pltpu.sync_copy(idx_hbm.at[pl.ds(base, CH)], idx_v)
            pltpu.async_copy(tab_hbm.at[idx_v], rows_v, sem).wait()
            pltpu.sync_copy(rows_v, out_hbm.at[pl.ds(base, CH)])

    return gk(table, idx)


def _sc_scatter_add(msg, dst, zeros_n):
    @functools.partial(
        pl.kernel,
        out_type=(
            jax.ShapeDtypeStruct((N, 64), jnp.float32),
            jax.ShapeDtypeStruct((N, 64), jnp.float32),
        ),
        mesh=_mesh(),
        scratch_types=[
            pltpu.VMEM((CH,), jnp.int32),
            pltpu.VMEM((CH, 64), jnp.float32),
            pltpu.VMEM_SHARED((N, 64), jnp.float32),
            pltpu.SemaphoreType.DMA,
        ],
        compiler_params=pltpu.CompilerParams(use_tc_tiling_on_sc=False),
        interpret=False,
    )
    def sk(msg_hbm, dst_hbm, z_hbm, out0, out1, idx_v, msg_v, acc, sem):
        c = lax.axis_index("c")
        s = lax.axis_index("s")

        @pl.when(s < 15)
        def _():
            r0 = pl.multiple_of(s * 624, 8)
            pltpu.sync_copy(z_hbm.at[pl.ds(r0, 624)], acc.at[pl.ds(r0, 624)])

        @pl.when(s == 15)
        def _():
            pltpu.sync_copy(z_hbm.at[pl.ds(9360, 640)], acc.at[pl.ds(9360, 640)])

        plsc.subcore_barrier()

        for j in range(CPW):
            base = pl.multiple_of(((s * NC + c) * CPW + j) * CH, CH)
            pltpu.sync_copy(dst_hbm.at[pl.ds(base, CH)], idx_v)
            pltpu.sync_copy(msg_hbm.at[pl.ds(base, CH)], msg_v)
            pltpu.sync_copy(msg_v, acc.at[idx_v], add=True)

        plsc.subcore_barrier()

        def dump(out_hbm):
            @pl.when(s < 15)
            def _():
                r0 = pl.multiple_of(s * 624, 8)
                pltpu.sync_copy(acc.at[pl.ds(r0, 624)], out_hbm.at[pl.ds(r0, 624)])

            @pl.when(s == 15)
            def _():
                pltpu.sync_copy(acc.at[pl.ds(9360, 640)], out_hbm.at[pl.ds(9360, 640)])

        @pl.when(c == 0)
        def _():
            dump(out0)

        @pl.when(c == 1)
        def _():
            dump(out1)

    return sk(msg, dst, zeros_n)


def _tc_edge_msgs(ea_pad, xs_pad, w2d, b_row, p_mat, q_mat, in_c, out_c):
    io = in_c * out_c
    grid = E_PAD // BE
    real_blocks = E // BE

    def body(ea_ref, xs_ref, w_ref, b_ref, p_ref, q_ref, o_ref):
        blk = pl.program_id(0)

        @pl.when(blk < real_blocks)
        def _():
            c0 = ea_ref[:, 0:1]
            c1 = ea_ref[:, 1:2]
            a = jnp.maximum(c0 * w_ref[0:1, :] + (c1 * w_ref[1:2, :] + b_ref[...]), 0.0)
            xe = jnp.dot(
                xs_ref[...].astype(jnp.bfloat16),
                p_ref[...],
                preferred_element_type=jnp.float32,
            )
            prod = (a * xe).astype(jnp.bfloat16)
            o_ref[...] = jnp.dot(prod, q_ref[...], preferred_element_type=jnp.float32)

        @pl.when(blk >= real_blocks)
        def _():
            o_ref[...] = jnp.zeros((BE, out_c), jnp.float32)

    return pl.pallas_call(
        body,
        grid=(grid,),
        in_specs=[
            pl.BlockSpec((BE, 2), lambda i: (i, 0)),
            pl.BlockSpec((BE, in_c), lambda i: (i, 0)),
            pl.BlockSpec((2, io), lambda i: (0, 0)),
            pl.BlockSpec((1, io), lambda i: (0, 0)),
            pl.BlockSpec((in_c, io), lambda i: (0, 0)),
            pl.BlockSpec((io, out_c), lambda i: (0, 0)),
        ],
        out_specs=pl.BlockSpec((BE, out_c), lambda i: (i, 0)),
        out_shape=jax.ShapeDtypeStruct((E_PAD, out_c), jnp.float32),
        interpret=False,
    )(ea_pad, xs_pad, w2d, b_row, p_mat, q_mat)


def _tc_combine(p0, p1, x_in, root, bias_row, in_c):
    def body(a_ref, b_ref, x_ref, r_ref, bias_ref, o_ref):
        o_ref[...] = (
            a_ref[...]
            + b_ref[...]
            + bias_ref[...]
            + jnp.dot(x_ref[...], r_ref[...], preferred_element_type=jnp.float32)
        )

    return pl.pallas_call(
        body,
        grid=(N // BN,),
        in_specs=[
            pl.BlockSpec((BN, 64), lambda i: (i, 0)),
            pl.BlockSpec((BN, 64), lambda i: (i, 0)),
            pl.BlockSpec((BN, in_c), lambda i: (i, 0)),
            pl.BlockSpec((in_c, 64), lambda i: (0, 0)),
            pl.BlockSpec((1, 64), lambda i: (0, 0)),
        ],
        out_specs=pl.BlockSpec((BN, 64), lambda i: (i, 0)),
        out_shape=jax.ShapeDtypeStruct((N, 64), jnp.float32),
        interpret=False,
    )(p0, p1, x_in, root, bias_row)


def _sel_mats(in_c, out_c):
    io = in_c * out_c
    j = jnp.arange(io)
    p_mat = (j[None, :] // out_c == jnp.arange(in_c)[:, None]).astype(jnp.bfloat16)
    q_mat = (j[:, None] % out_c == jnp.arange(out_c)[None, :]).astype(jnp.bfloat16)
    return p_mat, q_mat


def _layer(x_in, src_p, dst_p, ea_p, w, b, root, bias, zeros_n, in_c, out_c):
    xs = _sc_gather(x_in, src_p, in_c)
    p_mat, q_mat = _sel_mats(in_c, out_c)
    msg = _tc_edge_msgs(ea_p, xs, w, b.reshape(1, -1), p_mat, q_mat, in_c, out_c)
    part0, part1 = _sc_scatter_add(msg, dst_p, zeros_n)
    return _tc_combine(part0, part1, x_in, root, bias.reshape(1, -1), in_c)


def kernel(x, edge_index, edge_attr, W1, b1, root1, bias1, W2, b2, root2, bias2):
    x = x.astype(jnp.float32)
    ea = edge_attr.astype(jnp.float32)
    src = edge_index[0].astype(jnp.int32)
    dst = edge_index[1].astype(jnp.int32)

    pad = E_PAD - E
    src_p = jnp.concatenate([src, jnp.zeros((pad,), jnp.int32)])
    dst_p = jnp.concatenate([dst, jnp.zeros((pad,), jnp.int32)])
    ea_p = jnp.concatenate([ea, jnp.zeros((pad, 2), jnp.float32)])
    zeros_n = jnp.zeros((N, 64), jnp.float32)

    h = _layer(x, src_p, dst_p, ea_p, W1, b1, root1, bias1, zeros_n, IN1, OUT1)
    out = _layer(h, src_p, dst_p, ea_p, W2, b2, root2, bias2, zeros_n, IN2, OUT2)
    return out


# speedup vs baseline: 1.8799x; 1.0105x over previous
import functools

import jax
import jax.numpy as jnp
from jax import lax
from jax.experimental import pallas as pl
from jax.experimental.pallas import tpu as pltpu
from jax.experimental.pallas import tpu_sc as plsc

N = 10000
E = 160000
IN1, OUT1 = 8, 64
IN2, OUT2 = 64, 64

NC, NS = 2, 16
NW = NC * NS
CH = 128
CPW = 40
E_PAD = NW * CH * CPW

BE = 640
BN = 1000


def _mesh():
    return plsc.VectorSubcoreMesh(
        core_axis_name="c", subcore_axis_name="s", num_cores=NC, num_subcores=NS
    )


def _sc_gather(table, idx, d):
    @functools.partial(
        pl.kernel,
        out_type=jax.ShapeDtypeStruct((E_PAD, d), jnp.float32),
        mesh=_mesh(),
        scratch_types=[
            pltpu.VMEM((CH,), jnp.int32),
            pltpu.VMEM((CH, d), jnp.float32),
            pltpu.SemaphoreType.DMA,
        ],
        compiler_params=pltpu.CompilerParams(use_tc_tiling_on_sc=False),
        interpret=False,
    )
    def gk(tab_hbm, idx_hbm, out_hbm, idx_v, rows_v, sem):
        wid = lax.axis_index("s") * NC + lax.axis_index("c")
        for j in range(CPW):
            base = pl.multiple_of((wid * CPW + j) * CH, CH)
            pltpu.sync_copy(idx_hbm.at[pl.ds(base, CH)], idx_v)
            pltpu.async_copy(tab_hbm.at[idx_v], rows_v, sem).wait()
            pltpu.sync_copy(rows_v, out_hbm.at[pl.ds(base, CH)])

    return gk(table, idx)


def _sc_scatter_add(msg, dst, zeros_n):
    @functools.partial(
        pl.kernel,
        out_type=(
            jax.ShapeDtypeStruct((N, 64), jnp.float32),
            jax.ShapeDtypeStruct((N, 64), jnp.float32),
        ),
        mesh=_mesh(),
        scratch_types=[
            pltpu.VMEM((CH,), jnp.int32),
            pltpu.VMEM((CH, 64), jnp.float32),
            pltpu.VMEM_SHARED((N, 64), jnp.float32),
            pltpu.SemaphoreType.DMA,
        ],
        compiler_params=pltpu.CompilerParams(use_tc_tiling_on_sc=False),
        interpret=False,
    )
    def sk(msg_hbm, dst_hbm, z_hbm, out0, out1, idx_v, msg_v, acc, sem):
        c = lax.axis_index("c")
        s = lax.axis_index("s")

        @pl.when(s < 15)
        def _():
            r0 = pl.multiple_of(s * 624, 8)
            pltpu.sync_copy(z_hbm.at[pl.ds(r0, 624)], acc.at[pl.ds(r0, 624)])

        @pl.when(s == 15)
        def _():
            pltpu.sync_copy(z_hbm.at[pl.ds(9360, 640)], acc.at[pl.ds(9360, 640)])

        plsc.subcore_barrier()

        for j in range(CPW):
            base = pl.multiple_of(((s * NC + c) * CPW + j) * CH, CH)
            pltpu.sync_copy(dst_hbm.at[pl.ds(base, CH)], idx_v)
            pltpu.sync_copy(msg_hbm.at[pl.ds(base, CH)], msg_v)
            pltpu.sync_copy(msg_v, acc.at[idx_v], add=True)

        plsc.subcore_barrier()

        def dump(out_hbm):
            @pl.when(s < 15)
            def _():
                r0 = pl.multiple_of(s * 624, 8)
                pltpu.sync_copy(acc.at[pl.ds(r0, 624)], out_hbm.at[pl.ds(r0, 624)])

            @pl.when(s == 15)
            def _():
                pltpu.sync_copy(acc.at[pl.ds(9360, 640)], out_hbm.at[pl.ds(9360, 640)])

        @pl.when(c == 0)
        def _():
            dump(out0)

        @pl.when(c == 1)
        def _():
            dump(out1)

    return sk(msg, dst, zeros_n)


def _tc_edge_msgs(ea_pad, xs_pad, w2d, b_row, p_mat, q_mat, in_c, out_c):
    io = in_c * out_c
    grid = E_PAD // BE
    real_blocks = E // BE

    def body(ea_ref, xs_ref, w_ref, b_ref, p_ref, q_ref, o_ref):
        blk = pl.program_id(0)

        @pl.when(blk < real_blocks)
        def _():
            c0 = ea_ref[:, 0:1].astype(jnp.bfloat16)
            c1 = ea_ref[:, 1:2].astype(jnp.bfloat16)
            a = jnp.maximum(
                c0 * w_ref[0:1, :] + (c1 * w_ref[1:2, :] + b_ref[...]),
                jnp.bfloat16(0.0),
            )
            xe = jnp.dot(
                xs_ref[...].astype(jnp.bfloat16),
                p_ref[...],
                preferred_element_type=jnp.float32,
            ).astype(jnp.bfloat16)
            o_ref[...] = jnp.dot(a * xe, q_ref[...], preferred_element_type=jnp.float32)

        @pl.when(blk >= real_blocks)
        def _():
            o_ref[...] = jnp.zeros((BE, out_c), jnp.float32)

    return pl.pallas_call(
        body,
        grid=(grid,),
        in_specs=[
            pl.BlockSpec((BE, 2), lambda i: (i, 0)),
            pl.BlockSpec((BE, in_c), lambda i: (i, 0)),
            pl.BlockSpec((2, io), lambda i: (0, 0)),
            pl.BlockSpec((1, io), lambda i: (0, 0)),
            pl.BlockSpec((in_c, io), lambda i: (0, 0)),
            pl.BlockSpec((io, out_c), lambda i: (0, 0)),
        ],
        out_specs=pl.BlockSpec((BE, out_c), lambda i: (i, 0)),
        out_shape=jax.ShapeDtypeStruct((E_PAD, out_c), jnp.float32),
        interpret=False,
    )(ea_pad, xs_pad, w2d, b_row, p_mat, q_mat)


def _tc_combine(p0, p1, x_in, root, bias_row, in_c):
    def body(a_ref, b_ref, x_ref, r_ref, bias_ref, o_ref):
        o_ref[...] = (
            a_ref[...]
            + b_ref[...]
            + bias_ref[...]
            + jnp.dot(x_ref[...], r_ref[...], preferred_element_type=jnp.float32)
        )

    return pl.pallas_call(
        body,
        grid=(N // BN,),
        in_specs=[
            pl.BlockSpec((BN, 64), lambda i: (i, 0)),
            pl.BlockSpec((BN, 64), lambda i: (i, 0)),
            pl.BlockSpec((BN, in_c), lambda i: (i, 0)),
            pl.BlockSpec((in_c, 64), lambda i: (0, 0)),
            pl.BlockSpec((1, 64), lambda i: (0, 0)),
        ],
        out_specs=pl.BlockSpec((BN, 64), lambda i: (i, 0)),
        out_shape=jax.ShapeDtypeStruct((N, 64), jnp.float32),
        interpret=False,
    )(p0, p1, x_in, root, bias_row)


def _sel_mats(in_c, out_c):
    io = in_c * out_c
    j = jnp.arange(io)
    p_mat = (j[None, :] // out_c == jnp.arange(in_c)[:, None]).astype(jnp.bfloat16)
    q_mat = (j[:, None] % out_c == jnp.arange(out_c)[None, :]).astype(jnp.bfloat16)
    return p_mat, q_mat


def _layer(x_in, src_p, dst_p, ea_p, w, b, root, bias, zeros_n, in_c, out_c):
    xs = _sc_gather(x_in, src_p, in_c)
    p_mat, q_mat = _sel_mats(in_c, out_c)
    w_bf = w.astype(jnp.bfloat16)
    b_bf = b.reshape(1, -1).astype(jnp.bfloat16)
    msg = _tc_edge_msgs(ea_p, xs, w_bf, b_bf, p_mat, q_mat, in_c, out_c)
    part0, part1 = _sc_scatter_add(msg, dst_p, zeros_n)
    return _tc_combine(part0, part1, x_in, root, bias.reshape(1, -1), in_c)


def kernel(x, edge_index, edge_attr, W1, b1, root1, bias1, W2, b2, root2, bias2):
    x = x.astype(jnp.float32)
    ea = edge_attr.astype(jnp.float32)
    src = edge_index[0].astype(jnp.int32)
    dst = edge_index[1].astype(jnp.int32)

    pad = E_PAD - E
    src_p = jnp.concatenate([src, jnp.zeros((pad,), jnp.int32)])
    dst_p = jnp.concatenate([dst, jnp.zeros((pad,), jnp.int32)])
    ea_p = jnp.concatenate([ea, jnp.zeros((pad, 2), jnp.float32)])
    zeros_n = jnp.zeros((N, 64), jnp.float32)

    h = _layer(x, src_p, dst_p, ea_p, W1, b1, root1, bias1, zeros_n, IN1, OUT1)
    out = _layer(h, src_p, dst_p, ea_p, W2, b2, root2, bias2, zeros_n, IN2, OUT2)
    return out


# speedup vs baseline: 2.0411x; 1.0858x over previous
import functools

import jax
import jax.numpy as jnp
from jax import lax
from jax.experimental import pallas as pl
from jax.experimental.pallas import tpu as pltpu
from jax.experimental.pallas import tpu_sc as plsc

N = 10000
E = 160000
IN1, OUT1 = 8, 64
IN2, OUT2 = 64, 64

NC, NS = 2, 16
NW = NC * NS
CH = 128
CPW = 40
E_PAD = NW * CH * CPW

BE = 640
BN = 1000


def _mesh():
    return plsc.VectorSubcoreMesh(
        core_axis_name="c", subcore_axis_name="s", num_cores=NC, num_subcores=NS
    )


def _sc_gather(table, idx2, d):
    @functools.partial(
        pl.kernel,
        out_type=jax.ShapeDtypeStruct((E_PAD, d), jnp.float32),
        mesh=_mesh(),
        scratch_types=[
            pltpu.VMEM((CPW, CH), jnp.int32),
            pltpu.VMEM((CH, d), jnp.float32),
            pltpu.VMEM((CH, d), jnp.float32),
            pltpu.SemaphoreType.DMA,
            pltpu.SemaphoreType.DMA,
            pltpu.SemaphoreType.DMA,
            pltpu.SemaphoreType.DMA,
        ],
        compiler_params=pltpu.CompilerParams(use_tc_tiling_on_sc=False),
        interpret=False,
    )
    def gk(tab_hbm, idx_hbm, out_hbm, idx_v, r0, r1, g0, g1, o0, o1):
        wid = lax.axis_index("s") * NC + lax.axis_index("c")
        w0 = pl.multiple_of(wid * CPW, 8)
        pltpu.sync_copy(idx_hbm.at[pl.ds(w0, CPW)], idx_v)
        rows = (r0, r1)
        gsem = (g0, g1)
        osem = (o0, o1)
        gcp = [None, None]
        ocp = [None, None]
        gcp[0] = pltpu.async_copy(tab_hbm.at[idx_v.at[0]], rows[0], gsem[0])
        for j in range(CPW):
            b = j % 2
            nb = (j + 1) % 2
            if j + 1 < CPW:
                if ocp[nb] is not None:
                    ocp[nb].wait()
                gcp[nb] = pltpu.async_copy(
                    tab_hbm.at[idx_v.at[j + 1]], rows[nb], gsem[nb]
                )
            gcp[b].wait()
            base = pl.multiple_of((wid * CPW + j) * CH, CH)
            ocp[b] = pltpu.async_copy(rows[b], out_hbm.at[pl.ds(base, CH)], osem[b])
        ocp[0].wait()
        ocp[1].wait()

    return gk(table, idx2)


def _sc_scatter_add(msg, dst, zeros_n):
    @functools.partial(
        pl.kernel,
        out_type=(
            jax.ShapeDtypeStruct((N, 64), jnp.float32),
            jax.ShapeDtypeStruct((N, 64), jnp.float32),
        ),
        mesh=_mesh(),
        scratch_types=[
            pltpu.VMEM((CPW, CH), jnp.int32),
            pltpu.VMEM((4 * CH, 64), jnp.float32),
            pltpu.VMEM((4 * CH, 64), jnp.float32),
            pltpu.VMEM_SHARED((N, 64), jnp.float32),
            pltpu.SemaphoreType.DMA,
            pltpu.SemaphoreType.DMA,
        ],
        compiler_params=pltpu.CompilerParams(use_tc_tiling_on_sc=False),
        interpret=False,
    )
    def sk(msg_hbm, dst_hbm, z_hbm, out0, out1, idx_v, m0, m1, acc, s0, s1):
        c = lax.axis_index("c")
        s = lax.axis_index("s")

        @pl.when(s < 15)
        def _():
            r0 = pl.multiple_of(s * 624, 8)
            pltpu.sync_copy(z_hbm.at[pl.ds(r0, 624)], acc.at[pl.ds(r0, 624)])

        @pl.when(s == 15)
        def _():
            pltpu.sync_copy(z_hbm.at[pl.ds(9360, 640)], acc.at[pl.ds(9360, 640)])

        plsc.subcore_barrier()

        wid = s * NC + c
        w0 = pl.multiple_of(wid * CPW, 8)
        pltpu.sync_copy(dst_hbm.at[pl.ds(w0, CPW)], idx_v)
        bufs = (m0, m1)
        sems = (s0, s1)
        groups = CPW // 4
        mcp = [None, None]
        base0 = pl.multiple_of(wid * CPW * CH, CH)
        mcp[0] = pltpu.async_copy(msg_hbm.at[pl.ds(base0, 4 * CH)], bufs[0], sems[0])
        for g in range(groups):
            b = g % 2
            nb = (g + 1) % 2
            if g + 1 < groups:
                base = pl.multiple_of((wid * CPW + (g + 1) * 4) * CH, CH)
                mcp[nb] = pltpu.async_copy(
                    msg_hbm.at[pl.ds(base, 4 * CH)], bufs[nb], sems[nb]
                )
            mcp[b].wait()
            for k in range(4):
                pltpu.sync_copy(
                    bufs[b].at[pl.ds(k * CH, CH)],
                    acc.at[idx_v.at[g * 4 + k]],
                    add=True,
                )

        plsc.subcore_barrier()

        def dump(out_hbm):
            @pl.when(s < 15)
            def _():
                r0 = pl.multiple_of(s * 624, 8)
                pltpu.sync_copy(acc.at[pl.ds(r0, 624)], out_hbm.at[pl.ds(r0, 624)])

            @pl.when(s == 15)
            def _():
                pltpu.sync_copy(acc.at[pl.ds(9360, 640)], out_hbm.at[pl.ds(9360, 640)])

        @pl.when(c == 0)
        def _():
            dump(out0)

        @pl.when(c == 1)
        def _():
            dump(out1)

    return sk(msg, dst, zeros_n)


def _tc_edge_msgs(ea_pad, xs_pad, w2d, b_row, p_mat, q_mat, in_c, out_c):
    io = in_c * out_c
    grid = E_PAD // BE
    real_blocks = E // BE

    def body(ea_ref, xs_ref, w_ref, b_ref, p_ref, q_ref, o_ref):
        blk = pl.program_id(0)

        @pl.when(blk < real_blocks)
        def _():
            c0 = ea_ref[:, 0:1].astype(jnp.bfloat16)
            c1 = ea_ref[:, 1:2].astype(jnp.bfloat16)
            a = jnp.maximum(
                c0 * w_ref[0:1, :] + (c1 * w_ref[1:2, :] + b_ref[...]),
                jnp.bfloat16(0.0),
            )
            xe = jnp.dot(
                xs_ref[...].astype(jnp.bfloat16),
                p_ref[...],
                preferred_element_type=jnp.float32,
            ).astype(jnp.bfloat16)
            o_ref[...] = jnp.dot(a * xe, q_ref[...], preferred_element_type=jnp.float32)

        @pl.when(blk >= real_blocks)
        def _():
            o_ref[...] = jnp.zeros((BE, out_c), jnp.float32)

    return pl.pallas_call(
        body,
        grid=(grid,),
        in_specs=[
            pl.BlockSpec((BE, 2), lambda i: (i, 0)),
            pl.BlockSpec((BE, in_c), lambda i: (i, 0)),
            pl.BlockSpec((2, io), lambda i: (0, 0)),
            pl.BlockSpec((1, io), lambda i: (0, 0)),
            pl.BlockSpec((in_c, io), lambda i: (0, 0)),
            pl.BlockSpec((io, out_c), lambda i: (0, 0)),
        ],
        out_specs=pl.BlockSpec((BE, out_c), lambda i: (i, 0)),
        out_shape=jax.ShapeDtypeStruct((E_PAD, out_c), jnp.float32),
        interpret=False,
    )(ea_pad, xs_pad, w2d, b_row, p_mat, q_mat)


def _tc_combine(p0, p1, x_in, root, bias_row, in_c):
    def body(a_ref, b_ref, x_ref, r_ref, bias_ref, o_ref):
        o_ref[...] = (
            a_ref[...]
            + b_ref[...]
            + bias_ref[...]
            + jnp.dot(x_ref[...], r_ref[...], preferred_element_type=jnp.float32)
        )

    return pl.pallas_call(
        body,
        grid=(N // BN,),
        in_specs=[
            pl.BlockSpec((BN, 64), lambda i: (i, 0)),
            pl.BlockSpec((BN, 64), lambda i: (i, 0)),
            pl.BlockSpec((BN, in_c), lambda i: (i, 0)),
            pl.BlockSpec((in_c, 64), lambda i: (0, 0)),
            pl.BlockSpec((1, 64), lambda i: (0, 0)),
        ],
        out_specs=pl.BlockSpec((BN, 64), lambda i: (i, 0)),
        out_shape=jax.ShapeDtypeStruct((N, 64), jnp.float32),
        interpret=False,
    )(p0, p1, x_in, root, bias_row)


def _sel_mats(in_c, out_c):
    io = in_c * out_c
    j = jnp.arange(io)
    p_mat = (j[None, :] // out_c == jnp.arange(in_c)[:, None]).astype(jnp.bfloat16)
    q_mat = (j[:, None] % out_c == jnp.arange(out_c)[None, :]).astype(jnp.bfloat16)
    return p_mat, q_mat


def _layer(x_in, src_p, dst_p, ea_p, w, b, root, bias, zeros_n, in_c, out_c):
    xs = _sc_gather(x_in, src_p, in_c)
    p_mat, q_mat = _sel_mats(in_c, out_c)
    w_bf = w.astype(jnp.bfloat16)
    b_bf = b.reshape(1, -1).astype(jnp.bfloat16)
    msg = _tc_edge_msgs(ea_p, xs, w_bf, b_bf, p_mat, q_mat, in_c, out_c)
    part0, part1 = _sc_scatter_add(msg, dst_p, zeros_n)
    return _tc_combine(part0, part1, x_in, root, bias.reshape(1, -1), in_c)


def kernel(x, edge_index, edge_attr, W1, b1, root1, bias1, W2, b2, root2, bias2):
    x = x.astype(jnp.float32)
    ea = edge_attr.astype(jnp.float32)
    src = edge_index[0].astype(jnp.int32)
    dst = edge_index[1].astype(jnp.int32)

    pad = E_PAD - E
    src_p = jnp.concatenate([src, jnp.zeros((pad,), jnp.int32)]).reshape(E_PAD // CH, CH)
    dst_p = jnp.concatenate([dst, jnp.zeros((pad,), jnp.int32)]).reshape(E_PAD // CH, CH)
    ea_p = jnp.concatenate([ea, jnp.zeros((pad, 2), jnp.float32)])
    zeros_n = jnp.zeros((N, 64), jnp.float32)

    h = _layer(x, src_p, dst_p, ea_p, W1, b1, root1, bias1, zeros_n, IN1, OUT1)
    out = _layer(h, src_p, dst_p, ea_p, W2, b2, root2, bias2, zeros_n, IN2, OUT2)
    return out
